# baseline (speedup 1.0000x reference)
import functools

import jax
import jax.numpy as jnp
from jax import lax
from jax.experimental import pallas as pl
from jax.experimental.pallas import tpu as pltpu
from jax.experimental.pallas import tpu_sc as plsc

NC = 2
NS = 16
NW = NC * NS
XPAD = 128
OPAD_H = 56
OPAD_D = 128
NBUF = 8
LEAD = 4


@functools.cache
def _build(batch: int, hist: int, dim: int):
    rows_per_w = batch // NW
    assert rows_per_w % NBUF == 0
    mesh = plsc.VectorSubcoreMesh(core_axis_name="c", subcore_axis_name="s")

    @functools.partial(
        pl.kernel,
        out_type=jax.ShapeDtypeStruct((batch, OPAD_H, OPAD_D), jnp.float32),
        mesh=mesh,
        scratch_types=[
            pltpu.VMEM((rows_per_w, XPAD), jnp.int32),
            pltpu.VMEM((NBUF, OPAD_H, dim), jnp.float32),
            [pltpu.SemaphoreType.DMA] * NBUF,
            [pltpu.SemaphoreType.DMA] * NBUF,
        ],
        compiler_params=pltpu.CompilerParams(use_tc_tiling_on_sc=False),
    )
    def emb_kernel(x_hbm, w_hbm, out_hbm, idx_v, rows_v, gsems, osems):
        wid = lax.axis_index("s") * NC + lax.axis_index("c")
        base = wid * rows_per_w
        pltpu.sync_copy(x_hbm.at[pl.ds(base, rows_per_w), :], idx_v)

        def start_gather(k, b):
            pltpu.async_copy(
                w_hbm.at[idx_v.at[k, pl.ds(0, OPAD_H)]], rows_v.at[b], gsems[b]
            )

        def out_copy(k, b):
            return pltpu.make_async_copy(
                rows_v.at[b],
                out_hbm.at[base + k].at[pl.ds(0, OPAD_H), pl.ds(0, dim)],
                osems[b],
            )

        for g in range(LEAD):
            start_gather(g, g % NBUF)

        def group(grp, carry):
            for b in range(NBUF):
                k = grp * NBUF + b
                pltpu.make_async_copy(
                    w_hbm.at[idx_v.at[k, pl.ds(0, OPAD_H)]], rows_v.at[b],
                    gsems[b],
                ).wait()
                out_copy(k, b).start()
                kn = k + LEAD
                bn = (b + LEAD) % NBUF

                @pl.when(kn < rows_per_w)
                def _():
                    @pl.when(kn >= NBUF)
                    def _():
                        out_copy(kn - NBUF, bn).wait()

                    start_gather(kn, bn)

            return carry

        lax.fori_loop(0, rows_per_w // NBUF, group, 0, unroll=False)

        for b in range(NBUF):
            out_copy(rows_per_w - NBUF + b, b).wait()

    return emb_kernel


def kernel(x, weight):
    batch, hist = x.shape
    _, dim = weight.shape
    xpad = jnp.pad(x.astype(jnp.int32), ((0, 0), (0, XPAD - hist)))
    out = _build(batch, hist, dim)(xpad, weight)
    return out[:, :hist, :dim]
